# baseline (speedup 1.0000x reference)
import functools

import jax
import jax.numpy as jnp
from jax import lax
from jax.experimental import pallas as pl
from jax.experimental.pallas import tpu as pltpu
from jax.experimental.pallas import tpu_sc as plsc

N_NODES = 10000
N_EDGES = 320000
LAT = 16
FOLD = 8
EROWS = N_EDGES // FOLD
VROWS = N_NODES // FOLD
EBLK = 4000
IW = 100
IR = N_EDGES // IW
NTILES = 32
TPR = IR // NTILES
NPS = N_NODES // 16

_f32 = jnp.float32


def _ln_folded(h, bdo, g, b):
    mu = jnp.dot(h, bdo, preferred_element_type=_f32,
                 precision=lax.Precision.HIGHEST)
    d = h - mu
    var = jnp.dot(d * d, bdo, preferred_element_type=_f32,
                  precision=lax.Precision.HIGHEST)
    return d / jnp.sqrt(var + 1e-5) * g + b


def _enc_node_body(x_ref, w1_ref, b1_ref, w2_ref, b2_ref, bdo_ref, g_ref, bt_ref,
                   bb0_ref, bb1_ref, bc0_ref, bc1_ref, bp0_ref, b1n_ref,
                   v_ref, s_ref, d_ref, cs_ref, cd_ref, cp_ref):
    h1 = jnp.maximum(jnp.dot(x_ref[...], w1_ref[...], preferred_element_type=_f32) + b1_ref[...], 0.0)
    h2 = jnp.maximum(jnp.dot(h1, w2_ref[...], preferred_element_type=_f32) + b2_ref[...], 0.0)
    v0 = _ln_folded(h2, bdo_ref[...], g_ref[...], bt_ref[...])
    v_ref[...] = v0
    cs = jnp.dot(v0, bb0_ref[...], preferred_element_type=_f32)
    cd = jnp.dot(v0, bc0_ref[...], preferred_element_type=_f32)
    cs_ref[...] = cs
    cd_ref[...] = cd
    cp_ref[...] = jnp.dot(v0, bp0_ref[...], preferred_element_type=_f32) + b1n_ref[...]
    s_ref[...] = cs + jnp.dot(v0, bb1_ref[...], preferred_element_type=_f32)
    d_ref[...] = cd + jnp.dot(v0, bc1_ref[...], preferred_element_type=_f32)


def _enc_edge_body(ea_ref, w1_ref, b1_ref, w2_ref, b2_ref, bdo_ref, g_ref, bt_ref,
                   ba0_ref, b1e_ref, e_ref, f_ref):
    h1 = jnp.maximum(jnp.dot(ea_ref[...], w1_ref[...], preferred_element_type=_f32) + b1_ref[...], 0.0)
    h2 = jnp.maximum(jnp.dot(h1, w2_ref[...], preferred_element_type=_f32) + b2_ref[...], 0.0)
    e0 = _ln_folded(h2, bdo_ref[...], g_ref[...], bt_ref[...])
    e_ref[...] = e0
    f_ref[...] = jnp.dot(e0, ba0_ref[...], preferred_element_type=_f32) + b1e_ref[...]


def _edge_step_body(e_ref, f_ref, gs_ref, gd_ref, ba1_ref, w2_ref, b2_ref, bdo_ref,
                    lg_ref, lb_ref, eo_ref):
    h1 = jnp.maximum(
        jnp.dot(e_ref[...], ba1_ref[...], preferred_element_type=_f32)
        + f_ref[...] + (gs_ref[...] + gd_ref[...]), 0.0)
    h2 = jnp.maximum(jnp.dot(h1, w2_ref[...], preferred_element_type=_f32) + b2_ref[...], 0.0)
    eo_ref[...] = _ln_folded(h2, bdo_ref[...], lg_ref[...], lb_ref[...])


def _node_step_body(v_ref, pa_ref, pb_ref, inv_ref, cp_ref, cs_ref, cd_ref,
                    bp1_ref, bp2_ref, w2_ref, b2_ref, bdo_ref, lg_ref, lb_ref,
                    bb1_ref, bc1_ref, vo_ref, s_ref, d_ref):
    agg = (pa_ref[...] + pb_ref[...]) / inv_ref[...]
    aggp = jnp.dot(agg, bp2_ref[...], preferred_element_type=_f32)
    h1 = jnp.maximum(
        cp_ref[...] + jnp.dot(v_ref[...], bp1_ref[...], preferred_element_type=_f32) + aggp, 0.0)
    h2 = jnp.maximum(jnp.dot(h1, w2_ref[...], preferred_element_type=_f32) + b2_ref[...], 0.0)
    v_new = _ln_folded(h2, bdo_ref[...], lg_ref[...], lb_ref[...])
    vo_ref[...] = v_new
    s_ref[...] = cs_ref[...] + jnp.dot(v_new, bb1_ref[...], preferred_element_type=_f32)
    d_ref[...] = cd_ref[...] + jnp.dot(v_new, bc1_ref[...], preferred_element_type=_f32)


def _node_step1_body(v_ref, pa_ref, pb_ref, ca_ref, cb_ref, cp_ref, cs_ref, cd_ref,
                     bp1_ref, bp2_ref, w2_ref, b2_ref, bdo_ref, lg_ref, lb_ref,
                     bb1_ref, bc1_ref, vo_ref, s_ref, d_ref, inv_ref):
    inv = jnp.maximum(ca_ref[...] + cb_ref[...], 1.0)
    inv_ref[...] = inv
    agg = (pa_ref[...] + pb_ref[...]) / inv
    aggp = jnp.dot(agg, bp2_ref[...], preferred_element_type=_f32)
    h1 = jnp.maximum(
        cp_ref[...] + jnp.dot(v_ref[...], bp1_ref[...], preferred_element_type=_f32) + aggp, 0.0)
    h2 = jnp.maximum(jnp.dot(h1, w2_ref[...], preferred_element_type=_f32) + b2_ref[...], 0.0)
    v_new = _ln_folded(h2, bdo_ref[...], lg_ref[...], lb_ref[...])
    vo_ref[...] = v_new
    s_ref[...] = cs_ref[...] + jnp.dot(v_new, bb1_ref[...], preferred_element_type=_f32)
    d_ref[...] = cd_ref[...] + jnp.dot(v_new, bc1_ref[...], preferred_element_type=_f32)


def _dec_body(h_ref, w1_ref, b1_ref, w2_ref, b2_ref, bdo_ref, g_ref, bt_ref,
              wh_ref, bh_ref, o_ref):
    h1 = jnp.maximum(jnp.dot(h_ref[...], w1_ref[...], preferred_element_type=_f32) + b1_ref[...], 0.0)
    h2 = jnp.maximum(jnp.dot(h1, w2_ref[...], preferred_element_type=_f32) + b2_ref[...], 0.0)
    hn = _ln_folded(h2, bdo_ref[...], g_ref[...], bt_ref[...])
    o_ref[...] = jnp.dot(hn, wh_ref[...], preferred_element_type=_f32) + bh_ref[...]


def _edge_grid_call(body, ins_blocked, ins_full, n_out, out_minor=128):
    grid = EROWS // EBLK
    in_specs = ([pl.BlockSpec((EBLK, 128), lambda i: (i, 0)) for _ in ins_blocked] +
                [pl.BlockSpec(a.shape, lambda i, n=len(a.shape): (0,) * n) for a in ins_full])
    out_specs = [pl.BlockSpec((EBLK, out_minor), lambda i: (i, 0)) for _ in range(n_out)]
    out_shape = [jax.ShapeDtypeStruct((EROWS, out_minor), _f32) for _ in range(n_out)]
    if n_out == 1:
        out_specs, out_shape = out_specs[0], out_shape[0]
    return pl.pallas_call(
        body, grid=(grid,), in_specs=in_specs,
        out_specs=out_specs, out_shape=out_shape,
        compiler_params=pltpu.CompilerParams(
            dimension_semantics=("parallel",)),
    )(*ins_blocked, *ins_full)


def _whole_call(body, ins, out_shapes):
    out_shape = [jax.ShapeDtypeStruct(s, _f32) for s in out_shapes]
    return pl.pallas_call(body, out_shape=out_shape)(*ins)


def _sc_mesh():
    return plsc.VectorSubcoreMesh(core_axis_name="c", subcore_axis_name="s")


GRP = 10


@functools.cache
def _make_gather():
    @functools.partial(
        pl.kernel, mesh=_sc_mesh(),
        compiler_params=pltpu.CompilerParams(use_tc_tiling_on_sc=False),
        out_type=[jax.ShapeDtypeStruct((IR, IW, LAT), _f32),
                  jax.ShapeDtypeStruct((IR, IW, LAT), _f32)],
        scratch_types=[
            pltpu.VMEM((TPR, IW), jnp.int32),
            pltpu.VMEM((TPR, IW), jnp.int32),
            pltpu.VMEM((2, GRP, IW, LAT), _f32),
            pltpu.VMEM((2, GRP, IW, LAT), _f32),
            pltpu.SemaphoreType.DMA,
            pltpu.SemaphoreType.DMA,
            pltpu.SemaphoreType.DMA,
            pltpu.SemaphoreType.DMA,
        ],
    )
    def _gather_kernel(s_hbm, d_hbm, src_hbm, dst_hbm, gs_hbm, gd_hbm,
                       idx_s, idx_d, buf_s, buf_d, sem0, sem1, semw0, semw1):
        del semw0, semw1
        c = lax.axis_index("c")
        s = lax.axis_index("s")
        wid = s * 2 + c
        row0 = wid * TPR
        pltpu.sync_copy(src_hbm.at[wid], idx_s)
        pltpu.sync_copy(dst_hbm.at[wid], idx_d)
        sems = (sem0, sem1)
        ngrp = TPR // GRP

        def fire(g, slot):
            j0 = g * GRP
            for k in range(GRP):
                pltpu.async_copy(s_hbm.at[idx_s.at[j0 + k]],
                                 buf_s.at[slot, k], sems[slot])
                pltpu.async_copy(d_hbm.at[idx_d.at[j0 + k]],
                                 buf_d.at[slot, k], sems[slot])

        def drain_write(g, slot):
            j0 = g * GRP
            pltpu.make_async_copy(gs_hbm.at[pl.ds(row0 + j0, GRP)],
                                  buf_s.at[slot], sems[slot]).wait()
            pltpu.make_async_copy(gd_hbm.at[pl.ds(row0 + j0, GRP)],
                                  buf_d.at[slot], sems[slot]).wait()
            pltpu.sync_copy(buf_s.at[slot], gs_hbm.at[pl.ds(row0 + j0, GRP)])
            pltpu.sync_copy(buf_d.at[slot], gd_hbm.at[pl.ds(row0 + j0, GRP)])

        fire(0, 0)

        def group(i, carry):
            g = i * 2
            fire(g + 1, 1)
            drain_write(g, 0)
            fire(g + 2, 0)
            drain_write(g + 1, 1)
            return carry

        lax.fori_loop(0, ngrp // 2 - 1, group, 0)
        g_last = ngrp - 2
        fire(g_last + 1, 1)
        drain_write(g_last, 0)
        drain_write(g_last + 1, 1)

    return _gather_kernel


@functools.cache
def _make_scatter(with_cnt):
    out_type = [jax.ShapeDtypeStruct((2, N_NODES, LAT), _f32)]
    scratch = [
        pltpu.VMEM((TPR, IW), jnp.int32),
        pltpu.VMEM((2, GRP, IW, LAT), _f32),
        pltpu.VMEM_SHARED((N_NODES, LAT), _f32),
        pltpu.SemaphoreType.DMA,
        pltpu.SemaphoreType.DMA,
        pltpu.SemaphoreType.DMA,
    ]
    if with_cnt:
        out_type.append(jax.ShapeDtypeStruct((2, N_NODES, LAT), _f32))
        scratch.append(pltpu.VMEM((IW, LAT), _f32))
        scratch.append(pltpu.VMEM_SHARED((N_NODES, LAT), _f32))

    @functools.partial(pl.kernel, mesh=_sc_mesh(), out_type=out_type,
                       compiler_params=pltpu.CompilerParams(use_tc_tiling_on_sc=False),
                       scratch_types=scratch)
    def _scatter_kernel(m_hbm, dst_hbm, zeros_hbm, *refs):
        if with_cnt:
            p_hbm, c_hbm, idx_d, mbuf, acc, seml0, seml1, sema, ones, acc_c = refs
        else:
            p_hbm, idx_d, mbuf, acc, seml0, seml1, sema = refs
        c = lax.axis_index("c")
        s = lax.axis_index("s")
        wid = s * 2 + c
        row0 = wid * TPR
        semls = (seml0, seml1)
        ngrp = TPR // GRP

        @pl.when(s == 0)
        def _init():
            pltpu.sync_copy(zeros_hbm, acc)
            if with_cnt:
                pltpu.sync_copy(zeros_hbm, acc_c)

        if with_cnt:
            def fill(i, carry):
                ones[i] = jnp.full((LAT,), 1.0, _f32)
                return carry

            lax.fori_loop(0, IW, fill, 0)
        pltpu.sync_copy(dst_hbm.at[wid], idx_d)
        plsc.subcore_barrier()

        def load(g, slot):
            pltpu.async_copy(m_hbm.at[pl.ds(row0 + g * GRP, GRP)],
                             mbuf.at[slot], semls[slot])

        def scat(g, slot):
            j0 = g * GRP
            pltpu.make_async_copy(m_hbm.at[pl.ds(row0 + j0, GRP)],
                                  mbuf.at[slot], semls[slot]).wait()
            for k in range(GRP):
                pltpu.async_copy(mbuf.at[slot, k], acc.at[idx_d.at[j0 + k]],
                                 sema, add=True)
                if with_cnt:
                    pltpu.async_copy(ones, acc_c.at[idx_d.at[j0 + k]],
                                     sema, add=True)
            pltpu.make_async_copy(m_hbm.at[pl.ds(row0 + j0, GRP)],
                                  mbuf.at[slot], sema).wait()
            if with_cnt:
                pltpu.make_async_copy(m_hbm.at[pl.ds(row0 + j0, GRP)],
                                      mbuf.at[slot], sema).wait()

        load(0, 0)

        def group(i, carry):
            g = i * 2
            load(g + 1, 1)
            scat(g, 0)
            load(g + 2, 0)
            scat(g + 1, 1)
            return carry

        lax.fori_loop(0, ngrp // 2 - 1, group, 0)
        g_last = ngrp - 2
        load(g_last + 1, 1)
        scat(g_last, 0)
        scat(g_last + 1, 1)
        plsc.subcore_barrier()

        @pl.when(s == 0)
        def _readout():
            pltpu.sync_copy(acc, p_hbm.at[c])
            if with_cnt:
                pltpu.sync_copy(acc_c, c_hbm.at[c])

    return _scatter_kernel


def _bd(w):
    return jnp.kron(jnp.eye(FOLD, dtype=_f32), w)


def _tile_b(b):
    return jnp.tile(b, FOLD)[None, :]


def kernel(x, edge_attr, edge_index, enc_node, enc_edge, core_node, core_edge,
           dec_node, dec_edge, dec_node_out, dec_edge_out):
    ce, cn = core_edge, core_node
    w1e, b1e = ce['layers'][0]['W'], ce['layers'][0]['b']
    w2e, b2e = ce['layers'][1]['W'], ce['layers'][1]['b']
    w1n, b1n = cn['layers'][0]['W'], cn['layers'][0]['b']
    w2n, b2n = cn['layers'][1]['W'], cn['layers'][1]['b']
    a0, a1 = w1e[0:16], w1e[16:32]
    b0, b1 = w1e[32:48], w1e[48:64]
    c0, c1 = w1e[64:80], w1e[80:96]
    p0, p1, p2 = w1n[0:16], w1n[16:32], w1n[32:48]

    bdo = _bd(jnp.full((LAT, LAT), 1.0 / LAT, _f32))
    ba0, ba1 = _bd(a0), _bd(a1)
    bb0, bb1 = _bd(b0), _bd(b1)
    bc0, bc1 = _bd(c0), _bd(c1)
    bp0, bp1, bp2 = _bd(p0), _bd(p1), _bd(p2)
    bw2e, bw2n = _bd(w2e), _bd(w2n)
    t_b1e, t_b2e = _tile_b(b1e), _tile_b(b2e)
    t_b1n, t_b2n = _tile_b(b1n), _tile_b(b2n)
    t_lge, t_lbe = _tile_b(ce['ln_g']), _tile_b(ce['ln_b'])
    t_lgn, t_lbn = _tile_b(cn['ln_g']), _tile_b(cn['ln_b'])

    en, ee = enc_node, enc_edge
    w1_en_big = jnp.kron(jnp.eye(FOLD, dtype=_f32), en['layers'][0]['W'])
    x_r = x.reshape(VROWS, FOLD * 128)

    v_f, s_f, d_f, cs_f, cd_f, cp_f = _whole_call(
        _enc_node_body,
        [x_r, w1_en_big, _tile_b(en['layers'][0]['b']), _bd(en['layers'][1]['W']),
         _tile_b(en['layers'][1]['b']), bdo, _tile_b(en['ln_g']), _tile_b(en['ln_b']),
         bb0, bb1, bc0, bc1, bp0, t_b1n],
        [(VROWS, 128)] * 6)

    ea_r = edge_attr.reshape(EROWS, 128)
    e_f, f_f = _edge_grid_call(
        _enc_edge_body, [ea_r],
        [_bd(ee['layers'][0]['W']), _tile_b(ee['layers'][0]['b']),
         _bd(ee['layers'][1]['W']), _tile_b(ee['layers'][1]['b']), bdo,
         _tile_b(ee['ln_g']), _tile_b(ee['ln_b']), ba0, t_b1e], 2)

    src3d = edge_index[0].reshape(NTILES, TPR, IW)
    dst3d = edge_index[1].reshape(NTILES, TPR, IW)
    zeros_fold = jnp.zeros((N_NODES, LAT), _f32)

    _DBG_FAKE_GATHER = False
    _DBG_FAKE_SCATTER = False

    def _fake_gather(sf, df, sidx, didx):
        S = sf.reshape(N_NODES, LAT)
        D = df.reshape(N_NODES, LAT)
        return (S[sidx].reshape(EROWS, 128), D[didx].reshape(EROWS, 128))

    def _fake_scatter_cnt(mm, didx, z):
        m2 = mm.reshape(-1, LAT)
        p = jax.ops.segment_sum(m2, didx, num_segments=N_NODES)
        cc = jax.ops.segment_sum(jnp.ones_like(m2), didx, num_segments=N_NODES)
        zz = jnp.zeros_like(p)
        return (jnp.concatenate([p, zz], 0).reshape(2 * VROWS, 128),
                jnp.concatenate([cc, zz], 0).reshape(2 * VROWS, 128))

    def _fake_scatter(mm, didx, z):
        m2 = mm.reshape(-1, LAT)
        p = jax.ops.segment_sum(m2, didx, num_segments=N_NODES)
        return [jnp.concatenate([p, jnp.zeros_like(p)], 0).reshape(2 * VROWS, 128)]

    inv_f = None
    for step in range(4):
        if _DBG_FAKE_GATHER:
            gs, gd = _fake_gather(s_f, d_f, src3d.reshape(-1), dst3d.reshape(-1))
        else:
            gs, gd = _make_gather()(s_f.reshape(N_NODES, LAT),
                                    d_f.reshape(N_NODES, LAT), src3d, dst3d)
        e_f = _edge_grid_call(
            _edge_step_body, [e_f, f_f, gs.reshape(EROWS, 128), gd.reshape(EROWS, 128)],
            [ba1, bw2e, t_b2e, bdo, t_lge, t_lbe], 1)
        if step == 0:
            if _DBG_FAKE_SCATTER:
                p_part, c_part = _fake_scatter_cnt(e_f, dst3d.reshape(-1), zeros_fold)
            else:
                p_part, c_part = _make_scatter(True)(e_f.reshape(IR, IW, LAT), dst3d, zeros_fold)
            pf = p_part.reshape(2, VROWS, 128)
            cf = c_part.reshape(2, VROWS, 128)
            v_f, s_f, d_f, inv_f = _whole_call(
                _node_step1_body,
                [v_f, pf[0], pf[1], cf[0], cf[1], cp_f, cs_f, cd_f,
                 bp1, bp2, bw2n, t_b2n, bdo, t_lgn, t_lbn, bb1, bc1],
                [(VROWS, 128)] * 4)
        else:
            if _DBG_FAKE_SCATTER:
                (p_part,) = _fake_scatter(e_f, dst3d.reshape(-1), zeros_fold)
            else:
                (p_part,) = _make_scatter(False)(e_f.reshape(IR, IW, LAT), dst3d, zeros_fold)
            pf = p_part.reshape(2, VROWS, 128)
            v_f, s_f, d_f = _whole_call(
                _node_step_body,
                [v_f, pf[0], pf[1], inv_f, cp_f, cs_f, cd_f,
                 bp1, bp2, bw2n, t_b2n, bdo, t_lgn, t_lbn, bb1, bc1],
                [(VROWS, 128)] * 3)

    dn, de = dec_node, dec_edge
    v_out = _whole_call(
        _dec_body,
        [v_f, _bd(dn['layers'][0]['W']), _tile_b(dn['layers'][0]['b']),
         _bd(dn['layers'][1]['W']), _tile_b(dn['layers'][1]['b']), bdo,
         _tile_b(dn['ln_g']), _tile_b(dn['ln_b']),
         _bd(dec_node_out['W']), _tile_b(dec_node_out['b'])],
        [(VROWS, FOLD * 8)])[0]
    e_out = _edge_grid_call(
        _dec_body, [e_f],
        [_bd(de['layers'][0]['W']), _tile_b(de['layers'][0]['b']),
         _bd(de['layers'][1]['W']), _tile_b(de['layers'][1]['b']), bdo,
         _tile_b(de['ln_g']), _tile_b(de['ln_b']),
         _bd(dec_edge_out['W']), _tile_b(dec_edge_out['b'])],
        1, out_minor=FOLD * 8)
    return (v_out.reshape(N_NODES, 8), e_out.reshape(N_EDGES, 8))


# speedup vs baseline: 1.0008x; 1.0008x over previous
import functools

import jax
import jax.numpy as jnp
from jax import lax
from jax.experimental import pallas as pl
from jax.experimental.pallas import tpu as pltpu
from jax.experimental.pallas import tpu_sc as plsc

N_NODES = 10000
N_EDGES = 320000
LAT = 16
FOLD = 8
EROWS = N_EDGES // FOLD
VROWS = N_NODES // FOLD
EBLK = 4000
IW = 100
IR = N_EDGES // IW
NTILES = 32
TPR = IR // NTILES
NPS = N_NODES // 16

_f32 = jnp.float32


def _ln_folded(h, bdo, g, b):
    mu = jnp.dot(h, bdo, preferred_element_type=_f32,
                 precision=lax.Precision.HIGHEST)
    d = h - mu
    var = jnp.dot(d * d, bdo, preferred_element_type=_f32,
                  precision=lax.Precision.HIGHEST)
    return d / jnp.sqrt(var + 1e-5) * g + b


def _enc_node_body(x_ref, w1_ref, b1_ref, w2_ref, b2_ref, bdo_ref, g_ref, bt_ref,
                   bb0_ref, bb1_ref, bc0_ref, bc1_ref, bp0_ref, b1n_ref,
                   v_ref, s_ref, d_ref, cs_ref, cd_ref, cp_ref):
    h1 = jnp.maximum(jnp.dot(x_ref[...], w1_ref[...], preferred_element_type=_f32) + b1_ref[...], 0.0)
    h2 = jnp.maximum(jnp.dot(h1, w2_ref[...], preferred_element_type=_f32) + b2_ref[...], 0.0)
    v0 = _ln_folded(h2, bdo_ref[...], g_ref[...], bt_ref[...])
    v_ref[...] = v0
    cs = jnp.dot(v0, bb0_ref[...], preferred_element_type=_f32)
    cd = jnp.dot(v0, bc0_ref[...], preferred_element_type=_f32)
    cs_ref[...] = cs
    cd_ref[...] = cd
    cp_ref[...] = jnp.dot(v0, bp0_ref[...], preferred_element_type=_f32) + b1n_ref[...]
    s_ref[...] = cs + jnp.dot(v0, bb1_ref[...], preferred_element_type=_f32)
    d_ref[...] = cd + jnp.dot(v0, bc1_ref[...], preferred_element_type=_f32)


def _enc_edge_body(ea_ref, w1_ref, b1_ref, w2_ref, b2_ref, bdo_ref, g_ref, bt_ref,
                   ba0_ref, b1e_ref, e_ref, f_ref):
    h1 = jnp.maximum(jnp.dot(ea_ref[...], w1_ref[...], preferred_element_type=_f32) + b1_ref[...], 0.0)
    h2 = jnp.maximum(jnp.dot(h1, w2_ref[...], preferred_element_type=_f32) + b2_ref[...], 0.0)
    e0 = _ln_folded(h2, bdo_ref[...], g_ref[...], bt_ref[...])
    e_ref[...] = e0
    f_ref[...] = jnp.dot(e0, ba0_ref[...], preferred_element_type=_f32) + b1e_ref[...]


def _edge_step_body(e_ref, f_ref, gs_ref, gd_ref, ba1_ref, w2_ref, b2_ref, bdo_ref,
                    lg_ref, lb_ref, eo_ref):
    h1 = jnp.maximum(
        jnp.dot(e_ref[...], ba1_ref[...], preferred_element_type=_f32)
        + f_ref[...] + (gs_ref[...] + gd_ref[...]), 0.0)
    h2 = jnp.maximum(jnp.dot(h1, w2_ref[...], preferred_element_type=_f32) + b2_ref[...], 0.0)
    eo_ref[...] = _ln_folded(h2, bdo_ref[...], lg_ref[...], lb_ref[...])


def _node_step_body(v_ref, pa_ref, pb_ref, inv_ref, cp_ref, cs_ref, cd_ref,
                    bp1_ref, bp2_ref, w2_ref, b2_ref, bdo_ref, lg_ref, lb_ref,
                    bb1_ref, bc1_ref, vo_ref, s_ref, d_ref):
    agg = (pa_ref[...] + pb_ref[...]) / inv_ref[...]
    aggp = jnp.dot(agg, bp2_ref[...], preferred_element_type=_f32)
    h1 = jnp.maximum(
        cp_ref[...] + jnp.dot(v_ref[...], bp1_ref[...], preferred_element_type=_f32) + aggp, 0.0)
    h2 = jnp.maximum(jnp.dot(h1, w2_ref[...], preferred_element_type=_f32) + b2_ref[...], 0.0)
    v_new = _ln_folded(h2, bdo_ref[...], lg_ref[...], lb_ref[...])
    vo_ref[...] = v_new
    s_ref[...] = cs_ref[...] + jnp.dot(v_new, bb1_ref[...], preferred_element_type=_f32)
    d_ref[...] = cd_ref[...] + jnp.dot(v_new, bc1_ref[...], preferred_element_type=_f32)


def _node_step1_body(v_ref, pa_ref, pb_ref, ca_ref, cb_ref, cp_ref, cs_ref, cd_ref,
                     bp1_ref, bp2_ref, w2_ref, b2_ref, bdo_ref, lg_ref, lb_ref,
                     bb1_ref, bc1_ref, vo_ref, s_ref, d_ref, inv_ref):
    inv = jnp.maximum(ca_ref[...] + cb_ref[...], 1.0)
    inv_ref[...] = inv
    agg = (pa_ref[...] + pb_ref[...]) / inv
    aggp = jnp.dot(agg, bp2_ref[...], preferred_element_type=_f32)
    h1 = jnp.maximum(
        cp_ref[...] + jnp.dot(v_ref[...], bp1_ref[...], preferred_element_type=_f32) + aggp, 0.0)
    h2 = jnp.maximum(jnp.dot(h1, w2_ref[...], preferred_element_type=_f32) + b2_ref[...], 0.0)
    v_new = _ln_folded(h2, bdo_ref[...], lg_ref[...], lb_ref[...])
    vo_ref[...] = v_new
    s_ref[...] = cs_ref[...] + jnp.dot(v_new, bb1_ref[...], preferred_element_type=_f32)
    d_ref[...] = cd_ref[...] + jnp.dot(v_new, bc1_ref[...], preferred_element_type=_f32)


def _dec_body(h_ref, w1_ref, b1_ref, w2_ref, b2_ref, bdo_ref, g_ref, bt_ref,
              wh_ref, bh_ref, o_ref):
    h1 = jnp.maximum(jnp.dot(h_ref[...], w1_ref[...], preferred_element_type=_f32) + b1_ref[...], 0.0)
    h2 = jnp.maximum(jnp.dot(h1, w2_ref[...], preferred_element_type=_f32) + b2_ref[...], 0.0)
    hn = _ln_folded(h2, bdo_ref[...], g_ref[...], bt_ref[...])
    o_ref[...] = jnp.dot(hn, wh_ref[...], preferred_element_type=_f32) + bh_ref[...]


def _edge_grid_call(body, ins_blocked, ins_full, n_out, out_minor=128):
    grid = EROWS // EBLK
    in_specs = ([pl.BlockSpec((EBLK, 128), lambda i: (i, 0)) for _ in ins_blocked] +
                [pl.BlockSpec(a.shape, lambda i, n=len(a.shape): (0,) * n) for a in ins_full])
    out_specs = [pl.BlockSpec((EBLK, out_minor), lambda i: (i, 0)) for _ in range(n_out)]
    out_shape = [jax.ShapeDtypeStruct((EROWS, out_minor), _f32) for _ in range(n_out)]
    if n_out == 1:
        out_specs, out_shape = out_specs[0], out_shape[0]
    return pl.pallas_call(
        body, grid=(grid,), in_specs=in_specs,
        out_specs=out_specs, out_shape=out_shape,
        compiler_params=pltpu.CompilerParams(
            dimension_semantics=("parallel",)),
    )(*ins_blocked, *ins_full)


def _whole_call(body, ins, out_shapes):
    out_shape = [jax.ShapeDtypeStruct(s, _f32) for s in out_shapes]
    return pl.pallas_call(body, out_shape=out_shape)(*ins)


def _sc_mesh():
    return plsc.VectorSubcoreMesh(core_axis_name="c", subcore_axis_name="s")


GRP = 10


@functools.cache
def _make_gather():
    @functools.partial(
        pl.kernel, mesh=_sc_mesh(),
        compiler_params=pltpu.CompilerParams(use_tc_tiling_on_sc=False),
        out_type=[jax.ShapeDtypeStruct((IR, IW, LAT), _f32),
                  jax.ShapeDtypeStruct((IR, IW, LAT), _f32)],
        scratch_types=[
            pltpu.VMEM((TPR, IW), jnp.int32),
            pltpu.VMEM((TPR, IW), jnp.int32),
            pltpu.VMEM((2, GRP, IW, LAT), _f32),
            pltpu.VMEM((2, GRP, IW, LAT), _f32),
            pltpu.SemaphoreType.DMA,
            pltpu.SemaphoreType.DMA,
            pltpu.SemaphoreType.DMA,
            pltpu.SemaphoreType.DMA,
        ],
    )
    def _gather_kernel(s_hbm, d_hbm, src_hbm, dst_hbm, gs_hbm, gd_hbm,
                       idx_s, idx_d, buf_s, buf_d, sem0, sem1, semw0, semw1):
        del semw0, semw1
        c = lax.axis_index("c")
        s = lax.axis_index("s")
        wid = s * 2 + c
        row0 = wid * TPR
        pltpu.sync_copy(src_hbm.at[wid], idx_s)
        pltpu.sync_copy(dst_hbm.at[wid], idx_d)
        sems = (sem0, sem1)
        ngrp = TPR // GRP

        def fire(g, slot):
            j0 = g * GRP
            for k in range(GRP):
                pltpu.async_copy(s_hbm.at[idx_s.at[j0 + k]],
                                 buf_s.at[slot, k], sems[slot])
                pltpu.async_copy(d_hbm.at[idx_d.at[j0 + k]],
                                 buf_d.at[slot, k], sems[slot])

        def drain_write(g, slot):
            j0 = g * GRP
            pltpu.make_async_copy(gs_hbm.at[pl.ds(row0 + j0, GRP)],
                                  buf_s.at[slot], sems[slot]).wait()
            pltpu.make_async_copy(gd_hbm.at[pl.ds(row0 + j0, GRP)],
                                  buf_d.at[slot], sems[slot]).wait()
            pltpu.sync_copy(buf_s.at[slot], gs_hbm.at[pl.ds(row0 + j0, GRP)])
            pltpu.sync_copy(buf_d.at[slot], gd_hbm.at[pl.ds(row0 + j0, GRP)])

        fire(0, 0)

        def group(i, carry):
            g = i * 2
            fire(g + 1, 1)
            drain_write(g, 0)
            fire(g + 2, 0)
            drain_write(g + 1, 1)
            return carry

        lax.fori_loop(0, ngrp // 2 - 1, group, 0)
        g_last = ngrp - 2
        fire(g_last + 1, 1)
        drain_write(g_last, 0)
        drain_write(g_last + 1, 1)

    return _gather_kernel


@functools.cache
def _make_scatter(with_cnt):
    out_type = [jax.ShapeDtypeStruct((2, N_NODES, LAT), _f32)]
    scratch = [
        pltpu.VMEM((TPR, IW), jnp.int32),
        pltpu.VMEM((2, GRP, IW, LAT), _f32),
        pltpu.VMEM_SHARED((N_NODES, LAT), _f32),
        pltpu.SemaphoreType.DMA,
        pltpu.SemaphoreType.DMA,
        pltpu.SemaphoreType.DMA,
    ]
    if with_cnt:
        out_type.append(jax.ShapeDtypeStruct((2, N_NODES, LAT), _f32))
        scratch.append(pltpu.VMEM((IW, LAT), _f32))
        scratch.append(pltpu.VMEM_SHARED((N_NODES, LAT), _f32))

    @functools.partial(pl.kernel, mesh=_sc_mesh(), out_type=out_type,
                       compiler_params=pltpu.CompilerParams(use_tc_tiling_on_sc=False),
                       scratch_types=scratch)
    def _scatter_kernel(m_hbm, dst_hbm, zeros_hbm, *refs):
        if with_cnt:
            p_hbm, c_hbm, idx_d, mbuf, acc, seml0, seml1, sema, ones, acc_c = refs
        else:
            p_hbm, idx_d, mbuf, acc, seml0, seml1, sema = refs
        c = lax.axis_index("c")
        s = lax.axis_index("s")
        wid = s * 2 + c
        row0 = wid * TPR
        semls = (seml0, seml1)
        ngrp = TPR // GRP

        @pl.when(s == 0)
        def _init():
            pltpu.sync_copy(zeros_hbm, acc)
            if with_cnt:
                pltpu.sync_copy(zeros_hbm, acc_c)

        if with_cnt:
            def fill(i, carry):
                ones[i] = jnp.full((LAT,), 1.0, _f32)
                return carry

            lax.fori_loop(0, IW, fill, 0)
        pltpu.sync_copy(dst_hbm.at[wid], idx_d)
        plsc.subcore_barrier()

        def load(g, slot):
            pltpu.async_copy(m_hbm.at[pl.ds(row0 + g * GRP, GRP)],
                             mbuf.at[slot], semls[slot])

        def scat(g, slot):
            j0 = g * GRP
            pltpu.make_async_copy(m_hbm.at[pl.ds(row0 + j0, GRP)],
                                  mbuf.at[slot], semls[slot]).wait()
            for k in range(GRP):
                pltpu.async_copy(mbuf.at[slot, k], acc.at[idx_d.at[j0 + k]],
                                 sema, add=True)
                if with_cnt:
                    pltpu.async_copy(ones, acc_c.at[idx_d.at[j0 + k]],
                                     sema, add=True)
            pltpu.make_async_copy(m_hbm.at[pl.ds(row0 + j0, GRP)],
                                  mbuf.at[slot], sema).wait()
            if with_cnt:
                pltpu.make_async_copy(m_hbm.at[pl.ds(row0 + j0, GRP)],
                                      mbuf.at[slot], sema).wait()

        load(0, 0)

        def group(i, carry):
            g = i * 2
            load(g + 1, 1)
            scat(g, 0)
            load(g + 2, 0)
            scat(g + 1, 1)
            return carry

        lax.fori_loop(0, ngrp // 2 - 1, group, 0)
        g_last = ngrp - 2
        load(g_last + 1, 1)
        scat(g_last, 0)
        scat(g_last + 1, 1)
        plsc.subcore_barrier()

        @pl.when(s == 0)
        def _readout():
            pltpu.sync_copy(acc, p_hbm.at[c])
            if with_cnt:
                pltpu.sync_copy(acc_c, c_hbm.at[c])

    return _scatter_kernel


def _bd(w):
    return jnp.kron(jnp.eye(FOLD, dtype=_f32), w)


def _tile_b(b):
    return jnp.tile(b, FOLD)[None, :]


def kernel(x, edge_attr, edge_index, enc_node, enc_edge, core_node, core_edge,
           dec_node, dec_edge, dec_node_out, dec_edge_out):
    ce, cn = core_edge, core_node
    w1e, b1e = ce['layers'][0]['W'], ce['layers'][0]['b']
    w2e, b2e = ce['layers'][1]['W'], ce['layers'][1]['b']
    w1n, b1n = cn['layers'][0]['W'], cn['layers'][0]['b']
    w2n, b2n = cn['layers'][1]['W'], cn['layers'][1]['b']
    a0, a1 = w1e[0:16], w1e[16:32]
    b0, b1 = w1e[32:48], w1e[48:64]
    c0, c1 = w1e[64:80], w1e[80:96]
    p0, p1, p2 = w1n[0:16], w1n[16:32], w1n[32:48]

    bdo = _bd(jnp.full((LAT, LAT), 1.0 / LAT, _f32))
    ba0, ba1 = _bd(a0), _bd(a1)
    bb0, bb1 = _bd(b0), _bd(b1)
    bc0, bc1 = _bd(c0), _bd(c1)
    bp0, bp1, bp2 = _bd(p0), _bd(p1), _bd(p2)
    bw2e, bw2n = _bd(w2e), _bd(w2n)
    t_b1e, t_b2e = _tile_b(b1e), _tile_b(b2e)
    t_b1n, t_b2n = _tile_b(b1n), _tile_b(b2n)
    t_lge, t_lbe = _tile_b(ce['ln_g']), _tile_b(ce['ln_b'])
    t_lgn, t_lbn = _tile_b(cn['ln_g']), _tile_b(cn['ln_b'])

    en, ee = enc_node, enc_edge
    w1_en_big = jnp.kron(jnp.eye(FOLD, dtype=_f32), en['layers'][0]['W'])
    x_r = x.reshape(VROWS, FOLD * 128)

    v_f, s_f, d_f, cs_f, cd_f, cp_f = _whole_call(
        _enc_node_body,
        [x_r, w1_en_big, _tile_b(en['layers'][0]['b']), _bd(en['layers'][1]['W']),
         _tile_b(en['layers'][1]['b']), bdo, _tile_b(en['ln_g']), _tile_b(en['ln_b']),
         bb0, bb1, bc0, bc1, bp0, t_b1n],
        [(VROWS, 128)] * 6)

    ea_r = edge_attr.reshape(EROWS, 128)
    e_f, f_f = _edge_grid_call(
        _enc_edge_body, [ea_r],
        [_bd(ee['layers'][0]['W']), _tile_b(ee['layers'][0]['b']),
         _bd(ee['layers'][1]['W']), _tile_b(ee['layers'][1]['b']), bdo,
         _tile_b(ee['ln_g']), _tile_b(ee['ln_b']), ba0, t_b1e], 2)

    src3d = edge_index[0].reshape(NTILES, TPR, IW)
    dst3d = edge_index[1].reshape(NTILES, TPR, IW)
    zeros_fold = jnp.zeros((N_NODES, LAT), _f32)

    inv_f = None
    for step in range(4):
        gs, gd = _make_gather()(s_f.reshape(N_NODES, LAT),
                                d_f.reshape(N_NODES, LAT), src3d, dst3d)
        e_f = _edge_grid_call(
            _edge_step_body, [e_f, f_f, gs.reshape(EROWS, 128), gd.reshape(EROWS, 128)],
            [ba1, bw2e, t_b2e, bdo, t_lge, t_lbe], 1)
        if step == 0:
            p_part, c_part = _make_scatter(True)(e_f.reshape(IR, IW, LAT),
                                                 dst3d, zeros_fold)
            pf = p_part.reshape(2, VROWS, 128)
            cf = c_part.reshape(2, VROWS, 128)
            v_f, s_f, d_f, inv_f = _whole_call(
                _node_step1_body,
                [v_f, pf[0], pf[1], cf[0], cf[1], cp_f, cs_f, cd_f,
                 bp1, bp2, bw2n, t_b2n, bdo, t_lgn, t_lbn, bb1, bc1],
                [(VROWS, 128)] * 4)
        else:
            (p_part,) = _make_scatter(False)(e_f.reshape(IR, IW, LAT),
                                             dst3d, zeros_fold)
            pf = p_part.reshape(2, VROWS, 128)
            v_f, s_f, d_f = _whole_call(
                _node_step_body,
                [v_f, pf[0], pf[1], inv_f, cp_f, cs_f, cd_f,
                 bp1, bp2, bw2n, t_b2n, bdo, t_lgn, t_lbn, bb1, bc1],
                [(VROWS, 128)] * 3)

    dn, de = dec_node, dec_edge
    v_out = _whole_call(
        _dec_body,
        [v_f, _bd(dn['layers'][0]['W']), _tile_b(dn['layers'][0]['b']),
         _bd(dn['layers'][1]['W']), _tile_b(dn['layers'][1]['b']), bdo,
         _tile_b(dn['ln_g']), _tile_b(dn['ln_b']),
         _bd(dec_node_out['W']), _tile_b(dec_node_out['b'])],
        [(VROWS, FOLD * 8)])[0]
    e_out = _edge_grid_call(
        _dec_body, [e_f],
        [_bd(de['layers'][0]['W']), _tile_b(de['layers'][0]['b']),
         _bd(de['layers'][1]['W']), _tile_b(de['layers'][1]['b']), bdo,
         _tile_b(de['ln_g']), _tile_b(de['ln_b']),
         _bd(dec_edge_out['W']), _tile_b(dec_edge_out['b'])],
        1, out_minor=FOLD * 8)
    return (v_out.reshape(N_NODES, 8), e_out.reshape(N_EDGES, 8))
